# baseline (speedup 1.0000x reference)
import functools

import jax
import jax.numpy as jnp
from jax import lax
from jax.experimental import pallas as pl
from jax.experimental.pallas import tpu as pltpu
from jax.experimental.pallas import tpu_sc as plsc

F = 26
SV = 1001
E = 32
B = 4096
HIST = 200
DNN = 256
HID = 128
IN_DIM = F * E + E

V1 = 1000001
TXB = 2048
TXG = (V1 + TXB - 1) // TXB
VPAD = TXG * TXB

NC, NS, L = 2, 16, 16
NW = NC * NS
BPW = B // NW

HCB = 4
HCR = HCB * HIST
HNC = BPW // HCB
FPW = BPW * F
FNC = 8
FCR = FPW // FNC


def _sc_gather_pool(tflat, fidx, item_table, hist_flat):
    mesh = plsc.VectorSubcoreMesh(core_axis_name="c", subcore_axis_name="s")

    @functools.partial(
        pl.kernel,
        out_type=[
            jax.ShapeDtypeStruct((B * F, E), jnp.float32),
            jax.ShapeDtypeStruct((B, E), jnp.float32),
        ],
        mesh=mesh,
        compiler_params=pltpu.CompilerParams(use_tc_tiling_on_sc=False),
        scratch_types=[
            pltpu.VMEM((FPW,), jnp.int32),
            pltpu.VMEM((2, FCR, E), jnp.float32),
            pltpu.VMEM((BPW * HIST,), jnp.int32),
            pltpu.VMEM((2, HCR, E), jnp.float32),
            pltpu.VMEM((BPW, E), jnp.float32),
            [pltpu.SemaphoreType.DMA] * 2,
            [pltpu.SemaphoreType.DMA] * 2,
            [pltpu.SemaphoreType.DMA] * 2,
        ],
    )
    def k(tflat_hbm, fidx_hbm, item_hbm, hidx_hbm, fout_hbm, sout_hbm,
          fidx_v, frow_v, hidx_v, hrow_v, sout_v, hsem, fsem, wsem):
        wid = lax.axis_index("s") * NC + lax.axis_index("c")

        pltpu.sync_copy(fidx_hbm.at[pl.ds(wid * FPW, FPW)], fidx_v)
        for c in range(2):
            pltpu.async_copy(tflat_hbm.at[fidx_v.at[pl.ds(c * FCR, FCR)]],
                             frow_v.at[c], fsem[c])
        for c in range(FNC):
            t = c % 2
            pltpu.make_async_copy(tflat_hbm.at[pl.ds(0, FCR)], frow_v.at[t],
                                  fsem[t]).wait()
            pltpu.async_copy(
                frow_v.at[t],
                fout_hbm.at[pl.ds(wid * FPW + c * FCR, FCR)], wsem[t])
            if c + 2 < FNC:
                pltpu.make_async_copy(tflat_hbm.at[pl.ds(0, FCR)],
                                      frow_v.at[t], wsem[t]).wait()
                pltpu.async_copy(
                    tflat_hbm.at[fidx_v.at[pl.ds((c + 2) * FCR, FCR)]],
                    frow_v.at[t], fsem[t])
        for c in range(FNC - 2, FNC):
            pltpu.make_async_copy(tflat_hbm.at[pl.ds(0, FCR)],
                                  frow_v.at[c % 2], wsem[c % 2]).wait()

        pltpu.sync_copy(hidx_hbm.at[pl.ds(wid * BPW * HIST, BPW * HIST)],
                        hidx_v)

        def hstart(c, slot):
            pltpu.async_copy(item_hbm.at[hidx_v.at[pl.ds(c * HCR, HCR)]],
                             hrow_v.at[slot], hsem[slot])

        for s in range(2):
            hstart(s, s)

        def hbody(jj, carry):
            for s in range(2):
                c = jj * 2 + s
                pltpu.make_async_copy(item_hbm.at[pl.ds(0, HCR)],
                                      hrow_v.at[s], hsem[s]).wait()
                for bb in range(HCB):
                    r0 = bb * HIST

                    def rbody(h4, accs):
                        a0, a1, c0, c1 = accs
                        h = r0 + h4 * 4
                        a0 = a0 + hrow_v[s, h, pl.ds(0, L)]
                        a1 = a1 + hrow_v[s, h, pl.ds(L, L)]
                        c0 = c0 + hrow_v[s, h + 1, pl.ds(0, L)]
                        c1 = c1 + hrow_v[s, h + 1, pl.ds(L, L)]
                        a0 = a0 + hrow_v[s, h + 2, pl.ds(0, L)]
                        a1 = a1 + hrow_v[s, h + 2, pl.ds(L, L)]
                        c0 = c0 + hrow_v[s, h + 3, pl.ds(0, L)]
                        c1 = c1 + hrow_v[s, h + 3, pl.ds(L, L)]
                        return (a0, a1, c0, c1)

                    zero = jnp.zeros((L,), jnp.float32)
                    a0, a1, c0, c1 = lax.fori_loop(
                        0, HIST // 4, rbody, (zero, zero, zero, zero),
                        unroll=False)
                    b = c * HCB + bb
                    sout_v[b, pl.ds(0, L)] = a0 + c0
                    sout_v[b, pl.ds(L, L)] = a1 + c1
                hstart(jnp.minimum(c + 2, HNC - 1), s)
            return carry

        lax.fori_loop(0, HNC // 2, hbody, 0, unroll=False)
        for s in range(2):
            pltpu.make_async_copy(item_hbm.at[pl.ds(0, HCR)],
                                  hrow_v.at[s], hsem[s]).wait()
        pltpu.sync_copy(sout_v, sout_hbm.at[pl.ds(wid * BPW, BPW)])

    return k(tflat, fidx, item_table, hist_flat)


def _tx_body(in_ref, o_ref):
    y = jnp.transpose(in_ref[...], (1, 0))
    y3 = y.reshape(TXB // 4, 4, E)
    o_ref[...] = jnp.concatenate([y3[:, j, :] for j in range(4)], axis=1)


def _tx(itemT):
    return pl.pallas_call(
        _tx_body,
        grid=(TXG,),
        in_specs=[pl.BlockSpec((E, TXB), lambda i: (0, i))],
        out_specs=pl.BlockSpec((TXB // 4, 128), lambda i: (i, 0)),
        out_shape=jax.ShapeDtypeStruct((VPAD // 4, 128), jnp.float32),
    )(itemT)


def _mlp_body(f_ref, s_ref, h_ref, w1_ref, b1_ref, w2_ref, b2_ref, o_ref):
    x1 = f_ref[...]
    counts = jnp.sum((h_ref[...] != 0).astype(jnp.float32), axis=1,
                     keepdims=True)
    x2 = jnp.where(counts > 0.0, s_ref[...] / jnp.maximum(counts, 1.0), 0.0)
    h = jnp.dot(x1, w1_ref[0:F * E, :], preferred_element_type=jnp.float32,
                precision=lax.Precision.HIGHEST)
    h = h + jnp.dot(x2, w1_ref[F * E:IN_DIM, :],
                    preferred_element_type=jnp.float32,
                    precision=lax.Precision.HIGHEST)
    h = jnp.maximum(h + b1_ref[...], 0.0)
    o_ref[...] = jnp.dot(h, w2_ref[...], preferred_element_type=jnp.float32,
                         precision=lax.Precision.HIGHEST) + b2_ref[...]


def _mlp(femb, seq_sum, history, W1, b1, W2, b2):
    BM = 256
    return pl.pallas_call(
        _mlp_body,
        grid=(B // BM,),
        in_specs=[
            pl.BlockSpec((BM, F * E), lambda i: (i, 0)),
            pl.BlockSpec((BM, E), lambda i: (i, 0)),
            pl.BlockSpec((BM, HIST), lambda i: (i, 0)),
            pl.BlockSpec((IN_DIM, DNN), lambda i: (0, 0)),
            pl.BlockSpec((1, DNN), lambda i: (0, 0)),
            pl.BlockSpec((DNN, HID), lambda i: (0, 0)),
            pl.BlockSpec((1, HID), lambda i: (0, 0)),
        ],
        out_specs=pl.BlockSpec((BM, HID), lambda i: (i, 0)),
        out_shape=jax.ShapeDtypeStruct((B, HID), jnp.float32),
    )(femb, seq_sum, history, W1, b1[None, :], W2, b2[None, :])


def kernel(sparse_indices, history, sparse_tables, item_table, W1, b1, W2, b2):
    tflat = sparse_tables.reshape(F * SV, E)
    fidx = (sparse_indices.astype(jnp.int32)
            + (jnp.arange(F, dtype=jnp.int32) * SV)[None, :])
    fidx = fidx.reshape(B * F)
    hist_flat = history.astype(jnp.int32).reshape(B * HIST)
    item_lin = _tx(item_table.T).reshape(VPAD, E)
    frows, seq_sum = _sc_gather_pool(tflat, fidx, item_lin, hist_flat)
    femb = frows.reshape(B, F * E)
    return _mlp(femb, seq_sum, history, W1, b1, W2, b2)


# speedup vs baseline: 2.0707x; 2.0707x over previous
import functools

import jax
import jax.numpy as jnp
from jax import lax
from jax.experimental import pallas as pl
from jax.experimental.pallas import tpu as pltpu
from jax.experimental.pallas import tpu_sc as plsc

F = 26
SV = 1001
E = 32
B = 4096
HIST = 200
DNN = 256
HID = 128
IN_DIM = F * E + E

V1 = 1000001
TXB = 2048
TXG = (V1 + 4 * TXB - 1) // (4 * TXB)
VPAD = TXG * 4 * TXB
LASTB = (V1 + TXB - 1) // TXB - 1

NC, NS, L = 2, 16, 16
NW = NC * NS
BPW = B // NW

HCB = 4
HCR = HCB * HIST
HNC = BPW // HCB
FPW = BPW * F
FNC = 8
FCR = FPW // FNC


def _sc_gather_pool(tflat, fidx, item_table, hist_flat):
    mesh = plsc.VectorSubcoreMesh(core_axis_name="c", subcore_axis_name="s")

    @functools.partial(
        pl.kernel,
        out_type=[
            jax.ShapeDtypeStruct((B * F, E), jnp.float32),
            jax.ShapeDtypeStruct((B, E), jnp.float32),
        ],
        mesh=mesh,
        compiler_params=pltpu.CompilerParams(use_tc_tiling_on_sc=False),
        scratch_types=[
            pltpu.VMEM((FPW,), jnp.int32),
            pltpu.VMEM((2, FCR, E), jnp.float32),
            pltpu.VMEM((BPW * HIST,), jnp.int32),
            pltpu.VMEM((2, HCR, E), jnp.float32),
            pltpu.VMEM((BPW, E), jnp.float32),
            [pltpu.SemaphoreType.DMA] * 2,
            [pltpu.SemaphoreType.DMA] * 2,
            [pltpu.SemaphoreType.DMA] * 2,
        ],
    )
    def k(tflat_hbm, fidx_hbm, item_hbm, hidx_hbm, fout_hbm, sout_hbm,
          fidx_v, frow_v, hidx_v, hrow_v, sout_v, hsem, fsem, wsem):
        wid = lax.axis_index("s") * NC + lax.axis_index("c")

        pltpu.sync_copy(fidx_hbm.at[pl.ds(wid * FPW, FPW)], fidx_v)
        for c in range(2):
            pltpu.async_copy(tflat_hbm.at[fidx_v.at[pl.ds(c * FCR, FCR)]],
                             frow_v.at[c], fsem[c])
        for c in range(FNC):
            t = c % 2
            pltpu.make_async_copy(tflat_hbm.at[pl.ds(0, FCR)], frow_v.at[t],
                                  fsem[t]).wait()
            pltpu.async_copy(
                frow_v.at[t],
                fout_hbm.at[pl.ds(wid * FPW + c * FCR, FCR)], wsem[t])
            if c + 2 < FNC:
                pltpu.make_async_copy(tflat_hbm.at[pl.ds(0, FCR)],
                                      frow_v.at[t], wsem[t]).wait()
                pltpu.async_copy(
                    tflat_hbm.at[fidx_v.at[pl.ds((c + 2) * FCR, FCR)]],
                    frow_v.at[t], fsem[t])
        for c in range(FNC - 2, FNC):
            pltpu.make_async_copy(tflat_hbm.at[pl.ds(0, FCR)],
                                  frow_v.at[c % 2], wsem[c % 2]).wait()

        pltpu.sync_copy(hidx_hbm.at[pl.ds(wid * BPW * HIST, BPW * HIST)],
                        hidx_v)

        def hstart(c, slot):
            pltpu.async_copy(item_hbm.at[hidx_v.at[pl.ds(c * HCR, HCR)]],
                             hrow_v.at[slot], hsem[slot])

        for s in range(2):
            hstart(s, s)

        def hbody(jj, carry):
            for s in range(2):
                c = jj * 2 + s
                pltpu.make_async_copy(item_hbm.at[pl.ds(0, HCR)],
                                      hrow_v.at[s], hsem[s]).wait()
                for bb in range(HCB):
                    r0 = bb * HIST

                    def rbody(h4, accs):
                        a0, a1, c0, c1 = accs
                        h = r0 + h4 * 4
                        a0 = a0 + hrow_v[s, h, pl.ds(0, L)]
                        a1 = a1 + hrow_v[s, h, pl.ds(L, L)]
                        c0 = c0 + hrow_v[s, h + 1, pl.ds(0, L)]
                        c1 = c1 + hrow_v[s, h + 1, pl.ds(L, L)]
                        a0 = a0 + hrow_v[s, h + 2, pl.ds(0, L)]
                        a1 = a1 + hrow_v[s, h + 2, pl.ds(L, L)]
                        c0 = c0 + hrow_v[s, h + 3, pl.ds(0, L)]
                        c1 = c1 + hrow_v[s, h + 3, pl.ds(L, L)]
                        return (a0, a1, c0, c1)

                    zero = jnp.zeros((L,), jnp.float32)
                    a0, a1, c0, c1 = lax.fori_loop(
                        0, HIST // 4, rbody, (zero, zero, zero, zero),
                        unroll=False)
                    b = c * HCB + bb
                    sout_v[b, pl.ds(0, L)] = a0 + c0
                    sout_v[b, pl.ds(L, L)] = a1 + c1
                hstart(jnp.minimum(c + 2, HNC - 1), s)
            return carry

        lax.fori_loop(0, HNC // 2, hbody, 0, unroll=False)
        for s in range(2):
            pltpu.make_async_copy(item_hbm.at[pl.ds(0, HCR)],
                                  hrow_v.at[s], hsem[s]).wait()
        pltpu.sync_copy(sout_v, sout_hbm.at[pl.ds(wid * BPW, BPW)])

    return k(tflat, fidx, item_table, hist_flat)


def _tx_body(i0, i1, i2, i3, o_ref):
    s = jnp.concatenate([i0[...], i1[...], i2[...], i3[...]], axis=0)
    o_ref[...] = jnp.transpose(s, (1, 0))


def _tx(itemT):
    return pl.pallas_call(
        _tx_body,
        grid=(TXG,),
        in_specs=[
            pl.BlockSpec((E, TXB),
                         lambda i, a=a: (0, jnp.minimum(4 * i + a, LASTB)))
            for a in range(4)
        ],
        out_specs=pl.BlockSpec((TXB, 128), lambda i: (i, 0)),
        out_shape=jax.ShapeDtypeStruct((TXG * TXB, 128), jnp.float32),
    )(itemT, itemT, itemT, itemT)


def _mlp_body(f_ref, s_ref, h_ref, w1_ref, b1_ref, w2_ref, b2_ref, o_ref):
    x1 = f_ref[...]
    counts = jnp.sum((h_ref[...] != 0).astype(jnp.float32), axis=1,
                     keepdims=True)
    x2 = jnp.where(counts > 0.0, s_ref[...] / jnp.maximum(counts, 1.0), 0.0)
    h = jnp.dot(x1, w1_ref[0:F * E, :], preferred_element_type=jnp.float32,
                precision=lax.Precision.HIGHEST)
    h = h + jnp.dot(x2, w1_ref[F * E:IN_DIM, :],
                    preferred_element_type=jnp.float32,
                    precision=lax.Precision.HIGHEST)
    h = jnp.maximum(h + b1_ref[...], 0.0)
    o_ref[...] = jnp.dot(h, w2_ref[...], preferred_element_type=jnp.float32,
                         precision=lax.Precision.HIGHEST) + b2_ref[...]


def _mlp(femb, seq_sum, history, W1, b1, W2, b2):
    BM = 256
    return pl.pallas_call(
        _mlp_body,
        grid=(B // BM,),
        in_specs=[
            pl.BlockSpec((BM, F * E), lambda i: (i, 0)),
            pl.BlockSpec((BM, E), lambda i: (i, 0)),
            pl.BlockSpec((BM, HIST), lambda i: (i, 0)),
            pl.BlockSpec((IN_DIM, DNN), lambda i: (0, 0)),
            pl.BlockSpec((1, DNN), lambda i: (0, 0)),
            pl.BlockSpec((DNN, HID), lambda i: (0, 0)),
            pl.BlockSpec((1, HID), lambda i: (0, 0)),
        ],
        out_specs=pl.BlockSpec((BM, HID), lambda i: (i, 0)),
        out_shape=jax.ShapeDtypeStruct((B, HID), jnp.float32),
    )(femb, seq_sum, history, W1, b1[None, :], W2, b2[None, :])


def kernel(sparse_indices, history, sparse_tables, item_table, W1, b1, W2, b2):
    tflat = sparse_tables.reshape(F * SV, E)
    fidx = (sparse_indices.astype(jnp.int32)
            + (jnp.arange(F, dtype=jnp.int32) * SV)[None, :])
    fidx = fidx.reshape(B * F)
    h32 = history.astype(jnp.int32)
    hist_remap = (h32 & ~8191) | ((h32 & 2047) << 2) | ((h32 >> 11) & 3)
    hist_flat = hist_remap.reshape(B * HIST)
    item_lin = _tx(item_table.T).reshape(VPAD, E)
    frows, seq_sum = _sc_gather_pool(tflat, fidx, item_lin, hist_flat)
    femb = frows.reshape(B, F * E)
    return _mlp(femb, seq_sum, history, W1, b1, W2, b2)


# speedup vs baseline: 2.4987x; 1.2067x over previous
import functools

import jax
import jax.numpy as jnp
from jax import lax
from jax.experimental import pallas as pl
from jax.experimental.pallas import tpu as pltpu
from jax.experimental.pallas import tpu_sc as plsc

F = 26
SV = 1001
E = 32
B = 4096
HIST = 200
DNN = 256
HID = 128
IN_DIM = F * E + E

V1 = 1000001
TXB = 4096
TXG = (V1 + 4 * TXB - 1) // (4 * TXB)
VPAD = TXG * 4 * TXB
LASTB = (V1 + TXB - 1) // TXB - 1

NC, NS, L = 2, 16, 16
NW = NC * NS
BPW = B // NW

HCB = 4
HCR = HCB * HIST
HNC = BPW // HCB
FPW = BPW * F
FNC = 8
FCR = FPW // FNC


def _sc_gather_pool(tflat, fidx, item_table, hist_flat):
    mesh = plsc.VectorSubcoreMesh(core_axis_name="c", subcore_axis_name="s")

    @functools.partial(
        pl.kernel,
        out_type=[
            jax.ShapeDtypeStruct((B * F, E), jnp.float32),
            jax.ShapeDtypeStruct((B, E), jnp.float32),
        ],
        mesh=mesh,
        compiler_params=pltpu.CompilerParams(use_tc_tiling_on_sc=False),
        scratch_types=[
            pltpu.VMEM((FPW,), jnp.int32),
            pltpu.VMEM((2, FCR, E), jnp.float32),
            pltpu.VMEM((BPW * HIST,), jnp.int32),
            pltpu.VMEM((2, HCR, E), jnp.float32),
            pltpu.VMEM((BPW, E), jnp.float32),
            [pltpu.SemaphoreType.DMA] * 2,
            [pltpu.SemaphoreType.DMA] * 2,
            [pltpu.SemaphoreType.DMA] * 2,
        ],
    )
    def k(tflat_hbm, fidx_hbm, item_hbm, hidx_hbm, fout_hbm, sout_hbm,
          fidx_v, frow_v, hidx_v, hrow_v, sout_v, hsem, fsem, wsem):
        wid = lax.axis_index("s") * NC + lax.axis_index("c")

        pltpu.sync_copy(fidx_hbm.at[pl.ds(wid * FPW, FPW)], fidx_v)
        for c in range(2):
            pltpu.async_copy(tflat_hbm.at[fidx_v.at[pl.ds(c * FCR, FCR)]],
                             frow_v.at[c], fsem[c])
        for c in range(FNC):
            t = c % 2
            pltpu.make_async_copy(tflat_hbm.at[pl.ds(0, FCR)], frow_v.at[t],
                                  fsem[t]).wait()
            pltpu.async_copy(
                frow_v.at[t],
                fout_hbm.at[pl.ds(wid * FPW + c * FCR, FCR)], wsem[t])
            if c + 2 < FNC:
                pltpu.make_async_copy(tflat_hbm.at[pl.ds(0, FCR)],
                                      frow_v.at[t], wsem[t]).wait()
                pltpu.async_copy(
                    tflat_hbm.at[fidx_v.at[pl.ds((c + 2) * FCR, FCR)]],
                    frow_v.at[t], fsem[t])
        for c in range(FNC - 2, FNC):
            pltpu.make_async_copy(tflat_hbm.at[pl.ds(0, FCR)],
                                  frow_v.at[c % 2], wsem[c % 2]).wait()

        pltpu.sync_copy(hidx_hbm.at[pl.ds(wid * BPW * HIST, BPW * HIST)],
                        hidx_v)

        def hstart(c, slot):
            pltpu.async_copy(item_hbm.at[hidx_v.at[pl.ds(c * HCR, HCR)]],
                             hrow_v.at[slot], hsem[slot])

        for s in range(2):
            hstart(s, s)

        def hbody(jj, carry):
            for s in range(2):
                c = jj * 2 + s
                pltpu.make_async_copy(item_hbm.at[pl.ds(0, HCR)],
                                      hrow_v.at[s], hsem[s]).wait()
                for bb in range(HCB):
                    r0 = bb * HIST

                    def rbody(h4, accs):
                        a0, a1, c0, c1 = accs
                        h = r0 + h4 * 4
                        a0 = a0 + hrow_v[s, h, pl.ds(0, L)]
                        a1 = a1 + hrow_v[s, h, pl.ds(L, L)]
                        c0 = c0 + hrow_v[s, h + 1, pl.ds(0, L)]
                        c1 = c1 + hrow_v[s, h + 1, pl.ds(L, L)]
                        a0 = a0 + hrow_v[s, h + 2, pl.ds(0, L)]
                        a1 = a1 + hrow_v[s, h + 2, pl.ds(L, L)]
                        c0 = c0 + hrow_v[s, h + 3, pl.ds(0, L)]
                        c1 = c1 + hrow_v[s, h + 3, pl.ds(L, L)]
                        return (a0, a1, c0, c1)

                    zero = jnp.zeros((L,), jnp.float32)
                    a0, a1, c0, c1 = lax.fori_loop(
                        0, HIST // 4, rbody, (zero, zero, zero, zero),
                        unroll=False)
                    b = c * HCB + bb
                    sout_v[b, pl.ds(0, L)] = a0 + c0
                    sout_v[b, pl.ds(L, L)] = a1 + c1
                hstart(jnp.minimum(c + 2, HNC - 1), s)
            return carry

        lax.fori_loop(0, HNC // 2, hbody, 0, unroll=False)
        for s in range(2):
            pltpu.make_async_copy(item_hbm.at[pl.ds(0, HCR)],
                                  hrow_v.at[s], hsem[s]).wait()
        pltpu.sync_copy(sout_v, sout_hbm.at[pl.ds(wid * BPW, BPW)])

    return k(tflat, fidx, item_table, hist_flat)


def _tx_body(i0, i1, i2, i3, o_ref):
    s = jnp.concatenate([i0[...], i1[...], i2[...], i3[...]], axis=0)
    o_ref[...] = jnp.transpose(s, (1, 0))


def _tx(itemT):
    return pl.pallas_call(
        _tx_body,
        grid=(TXG,),
        in_specs=[
            pl.BlockSpec((E, TXB),
                         lambda i, a=a: (0, jnp.minimum(4 * i + a, LASTB)))
            for a in range(4)
        ],
        out_specs=pl.BlockSpec((TXB, 128), lambda i: (i, 0)),
        out_shape=jax.ShapeDtypeStruct((TXG * TXB, 128), jnp.float32),
    )(itemT, itemT, itemT, itemT)


def _mlp_body(f_ref, s_ref, h_ref, w1_ref, b1_ref, w2_ref, b2_ref, o_ref):
    x1 = f_ref[...]
    counts = jnp.sum((h_ref[...] != 0).astype(jnp.float32), axis=1,
                     keepdims=True)
    x2 = jnp.where(counts > 0.0, s_ref[...] / jnp.maximum(counts, 1.0), 0.0)
    h = jnp.dot(x1, w1_ref[0:F * E, :], preferred_element_type=jnp.float32)
    h = h + jnp.dot(x2, w1_ref[F * E:IN_DIM, :],
                    preferred_element_type=jnp.float32)
    h = jnp.maximum(h + b1_ref[...], 0.0)
    o_ref[...] = (jnp.dot(h, w2_ref[...], preferred_element_type=jnp.float32)
                  + b2_ref[...])


def _mlp(femb, seq_sum, history, W1, b1, W2, b2):
    BM = 256
    return pl.pallas_call(
        _mlp_body,
        grid=(B // BM,),
        in_specs=[
            pl.BlockSpec((BM, F * E), lambda i: (i, 0)),
            pl.BlockSpec((BM, E), lambda i: (i, 0)),
            pl.BlockSpec((BM, HIST), lambda i: (i, 0)),
            pl.BlockSpec((IN_DIM, DNN), lambda i: (0, 0)),
            pl.BlockSpec((1, DNN), lambda i: (0, 0)),
            pl.BlockSpec((DNN, HID), lambda i: (0, 0)),
            pl.BlockSpec((1, HID), lambda i: (0, 0)),
        ],
        out_specs=pl.BlockSpec((BM, HID), lambda i: (i, 0)),
        out_shape=jax.ShapeDtypeStruct((B, HID), jnp.float32),
    )(femb, seq_sum, history, W1, b1[None, :], W2, b2[None, :])


def kernel(sparse_indices, history, sparse_tables, item_table, W1, b1, W2, b2):
    tflat = sparse_tables.reshape(F * SV, E)
    fidx = (sparse_indices.astype(jnp.int32)
            + (jnp.arange(F, dtype=jnp.int32) * SV)[None, :])
    fidx = fidx.reshape(B * F)
    h32 = history.astype(jnp.int32)
    hist_remap = ((h32 & ~(4 * TXB - 1)) | ((h32 & (TXB - 1)) << 2)
                  | ((h32 // TXB) & 3))
    hist_flat = hist_remap.reshape(B * HIST)
    item_lin = _tx(item_table.T).reshape(VPAD, E)
    frows, seq_sum = _sc_gather_pool(tflat, fidx, item_lin, hist_flat)
    femb = frows.reshape(B, F * E)
    return _mlp(femb, seq_sum, history, W1, b1, W2, b2)
